# baseline (device time: 89344 ns/iter reference)
import jax
import jax.numpy as jnp
from jax import lax
from jax.experimental import pallas as pl
from jax.experimental.pallas import tpu as pltpu

N_DEV = 16
SHIFT = 11
SEND_WINDOW = 3


def kernel(x, w_mat, scale_x, scale_w):
    m_per, k = x.shape
    n = w_mat.shape[1]
    n_per = n // N_DEV
    m_tot = N_DEV * m_per

    def body(x_ref, w_ref, sx_ref, sw_ref, out_ref, y_ref, rx_ref,
             send_sems, recv_sems):
        me = lax.axis_index("i")
        scale = sx_ref[0] * sw_ref[0]
        deq = scale * float(1 << SHIFT)

        x_blk = x_ref[...]

        for d in range(1, N_DEV):
            q = lax.rem(me + d, N_DEV)
            acc = jax.lax.dot_general(
                x_blk,
                w_ref[:, pl.ds(q * n_per, n_per)],
                dimension_numbers=(((1,), (0,)), ((), ())),
                preferred_element_type=jnp.int32,
            )
            y_ref[d] = jnp.right_shift(
                acc + (1 << (SHIFT - 1)), SHIFT
            ).astype(jnp.int16)
            rdma = pltpu.make_async_remote_copy(
                src_ref=y_ref.at[d],
                dst_ref=rx_ref.at[d],
                send_sem=send_sems.at[d],
                recv_sem=recv_sems.at[d],
                device_id=(q,),
                device_id_type=pl.DeviceIdType.MESH,
            )
            rdma.start()
            if d > SEND_WINDOW:
                old = d - SEND_WINDOW
                drain = pltpu.make_async_remote_copy(
                    src_ref=y_ref.at[old],
                    dst_ref=rx_ref.at[old],
                    send_sem=send_sems.at[old],
                    recv_sem=recv_sems.at[old],
                    device_id=(0,),
                    device_id_type=pl.DeviceIdType.MESH,
                )
                drain.wait_send()

        acc = jax.lax.dot_general(
            x_blk,
            w_ref[:, pl.ds(me * n_per, n_per)],
            dimension_numbers=(((1,), (0,)), ((), ())),
            preferred_element_type=jnp.int32,
        )
        out_ref[pl.ds(me * m_per, m_per), :] = jnp.maximum(
            acc.astype(jnp.float32) * scale, 0.0
        )

        for d in range(1, N_DEV):
            src = lax.rem(me - d + N_DEV, N_DEV)
            recv = pltpu.make_async_remote_copy(
                src_ref=y_ref.at[d],
                dst_ref=rx_ref.at[d],
                send_sem=send_sems.at[d],
                recv_sem=recv_sems.at[d],
                device_id=(src,),
                device_id_type=pl.DeviceIdType.MESH,
            )
            recv.wait_recv()
            out_ref[pl.ds(src * m_per, m_per), :] = jnp.maximum(
                rx_ref[d].astype(jnp.float32) * deq, 0.0
            )

        for d in range(N_DEV - SEND_WINDOW, N_DEV):
            send = pltpu.make_async_remote_copy(
                src_ref=y_ref.at[d],
                dst_ref=rx_ref.at[d],
                send_sem=send_sems.at[d],
                recv_sem=recv_sems.at[d],
                device_id=(0,),
                device_id_type=pl.DeviceIdType.MESH,
            )
            send.wait_send()

    return pl.pallas_call(
        body,
        out_shape=jax.ShapeDtypeStruct((m_tot, n_per), jnp.float32),
        in_specs=[
            pl.BlockSpec(memory_space=pltpu.VMEM),
            pl.BlockSpec(memory_space=pltpu.VMEM),
            pl.BlockSpec(memory_space=pltpu.SMEM),
            pl.BlockSpec(memory_space=pltpu.SMEM),
        ],
        out_specs=pl.BlockSpec(memory_space=pltpu.VMEM),
        scratch_shapes=[
            pltpu.VMEM((N_DEV, m_per, n_per), jnp.int16),
            pltpu.VMEM((N_DEV, m_per, n_per), jnp.int16),
            pltpu.SemaphoreType.DMA((N_DEV,)),
            pltpu.SemaphoreType.DMA((N_DEV,)),
        ],
        compiler_params=pltpu.CompilerParams(
            vmem_limit_bytes=100 * 1024 * 1024,
        ),
    )(x, w_mat, scale_x, scale_w)


# device time: 58970 ns/iter; 1.5151x vs baseline; 1.5151x over previous
import jax
import jax.numpy as jnp
from jax import lax
from jax.experimental import pallas as pl
from jax.experimental.pallas import tpu as pltpu

N_DEV = 16
SHIFT = 13


def kernel(x, w_mat, scale_x, scale_w):
    m_per, k = x.shape
    n = w_mat.shape[1]
    n_per = n // N_DEV
    m_tot = N_DEV * m_per

    def body(x_ref, w_ref, sx_ref, sw_ref, out_ref, y_ref, rx_ref,
             send_sems, recv_sems):
        me = lax.axis_index("i")
        scale = sx_ref[0] * sw_ref[0]
        deq = scale * float(1 << SHIFT)

        x_blk = x_ref[...]

        for d in range(1, N_DEV):
            q = lax.rem(me + d, N_DEV)
            acc = jax.lax.dot_general(
                x_blk,
                w_ref[:, pl.ds(q * n_per, n_per)],
                dimension_numbers=(((1,), (0,)), ((), ())),
                preferred_element_type=jnp.int32,
            )
            v = jnp.right_shift(
                jnp.maximum(acc, 0) + (1 << (SHIFT - 1)), SHIFT
            )
            y_ref[d] = (jnp.minimum(v, 255) - 128).astype(jnp.int8)
            rdma = pltpu.make_async_remote_copy(
                src_ref=y_ref.at[d],
                dst_ref=rx_ref.at[d],
                send_sem=send_sems.at[d],
                recv_sem=recv_sems.at[d],
                device_id=(q,),
                device_id_type=pl.DeviceIdType.MESH,
            )
            rdma.start()

        acc = jax.lax.dot_general(
            x_blk,
            w_ref[:, pl.ds(me * n_per, n_per)],
            dimension_numbers=(((1,), (0,)), ((), ())),
            preferred_element_type=jnp.int32,
        )
        out_ref[pl.ds(me * m_per, m_per), :] = jnp.maximum(
            acc.astype(jnp.float32) * scale, 0.0
        )

        for d in range(1, N_DEV):
            src = lax.rem(me - d + N_DEV, N_DEV)
            recv = pltpu.make_async_remote_copy(
                src_ref=y_ref.at[d],
                dst_ref=rx_ref.at[d],
                send_sem=send_sems.at[d],
                recv_sem=recv_sems.at[d],
                device_id=(src,),
                device_id_type=pl.DeviceIdType.MESH,
            )
            recv.wait_recv()
            out_ref[pl.ds(src * m_per, m_per), :] = (
                (rx_ref[d].astype(jnp.int32) + 128).astype(jnp.float32) * deq
            )

        for d in range(1, N_DEV):
            send = pltpu.make_async_remote_copy(
                src_ref=y_ref.at[d],
                dst_ref=rx_ref.at[d],
                send_sem=send_sems.at[d],
                recv_sem=recv_sems.at[d],
                device_id=(0,),
                device_id_type=pl.DeviceIdType.MESH,
            )
            send.wait_send()

    return pl.pallas_call(
        body,
        out_shape=jax.ShapeDtypeStruct((m_tot, n_per), jnp.float32),
        in_specs=[
            pl.BlockSpec(memory_space=pltpu.VMEM),
            pl.BlockSpec(memory_space=pltpu.VMEM),
            pl.BlockSpec(memory_space=pltpu.SMEM),
            pl.BlockSpec(memory_space=pltpu.SMEM),
        ],
        out_specs=pl.BlockSpec(memory_space=pltpu.VMEM),
        scratch_shapes=[
            pltpu.VMEM((N_DEV, m_per, n_per), jnp.int8),
            pltpu.VMEM((N_DEV, m_per, n_per), jnp.int8),
            pltpu.SemaphoreType.DMA((N_DEV,)),
            pltpu.SemaphoreType.DMA((N_DEV,)),
        ],
        compiler_params=pltpu.CompilerParams(
            vmem_limit_bytes=100 * 1024 * 1024,
        ),
    )(x, w_mat, scale_x, scale_w)


# device time: 58391 ns/iter; 1.5301x vs baseline; 1.0099x over previous
import jax
import jax.numpy as jnp
from jax import lax
from jax.experimental import pallas as pl
from jax.experimental.pallas import tpu as pltpu

N_DEV = 16
SHIFT = 13
GROUPS = 4


def kernel(x, w_mat, scale_x, scale_w):
    m_per, k = x.shape
    n = w_mat.shape[1]
    n_per = n // N_DEV
    m_tot = N_DEV * m_per
    g = N_DEV // GROUPS
    n_chunk = n // GROUPS

    def body(x_ref, w_ref, sx_ref, sw_ref, out_ref, y_ref, rx_ref,
             send_sems, recv_sems):
        me = lax.axis_index("i")
        scale = sx_ref[0] * sw_ref[0]
        deq = scale * float(1 << SHIFT)

        x_blk = x_ref[...]

        for c in range(GROUPS):
            acc = jax.lax.dot_general(
                x_blk,
                w_ref[:, c * n_chunk:(c + 1) * n_chunk],
                dimension_numbers=(((1,), (0,)), ((), ())),
                preferred_element_type=jnp.int32,
            )
            v = jnp.right_shift(
                jnp.maximum(acc, 0) + (1 << (SHIFT - 1)), SHIFT
            )
            y_ref[:, c * n_chunk:(c + 1) * n_chunk] = (
                jnp.minimum(v, 255) - 128
            ).astype(jnp.int8)
            for j in range(g):
                q = c * g + j
                d = lax.rem(q - me + N_DEV, N_DEV)
                rdma = pltpu.make_async_remote_copy(
                    src_ref=y_ref.at[:, pl.ds(q * n_per, n_per)],
                    dst_ref=rx_ref.at[d],
                    send_sem=send_sems.at[d],
                    recv_sem=recv_sems.at[d],
                    device_id=(q,),
                    device_id_type=pl.DeviceIdType.MESH,
                )
                rdma.start()

        for d in range(N_DEV):
            src = lax.rem(me - d + N_DEV, N_DEV)
            recv = pltpu.make_async_remote_copy(
                src_ref=y_ref.at[:, pl.ds(0, n_per)],
                dst_ref=rx_ref.at[d],
                send_sem=send_sems.at[d],
                recv_sem=recv_sems.at[d],
                device_id=(src,),
                device_id_type=pl.DeviceIdType.MESH,
            )
            recv.wait_recv()
            out_ref[pl.ds(src * m_per, m_per), :] = (
                (rx_ref[d].astype(jnp.int32) + 128).astype(jnp.float32) * deq
            )

        for d in range(N_DEV):
            send = pltpu.make_async_remote_copy(
                src_ref=y_ref.at[:, pl.ds(0, n_per)],
                dst_ref=rx_ref.at[d],
                send_sem=send_sems.at[d],
                recv_sem=recv_sems.at[d],
                device_id=(0,),
                device_id_type=pl.DeviceIdType.MESH,
            )
            send.wait_send()

    return pl.pallas_call(
        body,
        out_shape=jax.ShapeDtypeStruct((m_tot, n_per), jnp.float32),
        in_specs=[
            pl.BlockSpec(memory_space=pltpu.VMEM),
            pl.BlockSpec(memory_space=pltpu.VMEM),
            pl.BlockSpec(memory_space=pltpu.SMEM),
            pl.BlockSpec(memory_space=pltpu.SMEM),
        ],
        out_specs=pl.BlockSpec(memory_space=pltpu.VMEM),
        scratch_shapes=[
            pltpu.VMEM((m_per, n), jnp.int8),
            pltpu.VMEM((N_DEV, m_per, n_per), jnp.int8),
            pltpu.SemaphoreType.DMA((N_DEV,)),
            pltpu.SemaphoreType.DMA((N_DEV,)),
        ],
        compiler_params=pltpu.CompilerParams(
            vmem_limit_bytes=100 * 1024 * 1024,
        ),
    )(x, w_mat, scale_x, scale_w)


# device time: 53863 ns/iter; 1.6587x vs baseline; 1.0841x over previous
import jax
import jax.numpy as jnp
from jax import lax
from jax.experimental import pallas as pl
from jax.experimental.pallas import tpu as pltpu

N_DEV = 16
SHIFT = 13
GROUPS = 16


def kernel(x, w_mat, scale_x, scale_w):
    m_per, k = x.shape
    n = w_mat.shape[1]
    n_per = n // N_DEV
    m_tot = N_DEV * m_per
    g = N_DEV // GROUPS
    n_chunk = n // GROUPS

    def body(x_ref, w_ref, sx_ref, sw_ref, out_ref, y_ref, rx_ref,
             send_sems, recv_sems):
        me = lax.axis_index("i")
        scale = sx_ref[0] * sw_ref[0]
        deq = scale * float(1 << SHIFT)

        x_blk = x_ref[...]

        for c in range(GROUPS):
            acc = jax.lax.dot_general(
                x_blk,
                w_ref[:, c * n_chunk:(c + 1) * n_chunk],
                dimension_numbers=(((1,), (0,)), ((), ())),
                preferred_element_type=jnp.int32,
            )
            v = jnp.right_shift(
                jnp.maximum(acc, 0) + (1 << (SHIFT - 1)), SHIFT
            )
            y_ref[:, c * n_chunk:(c + 1) * n_chunk] = (
                jnp.minimum(v, 255) - 128
            ).astype(jnp.int8)
            for j in range(g):
                q = c * g + j
                d = lax.rem(q - me + N_DEV, N_DEV)

                @pl.when(d != 0)
                def _():
                    rdma = pltpu.make_async_remote_copy(
                        src_ref=y_ref.at[:, pl.ds(q * n_per, n_per)],
                        dst_ref=rx_ref.at[d],
                        send_sem=send_sems.at[d],
                        recv_sem=recv_sems.at[d],
                        device_id=(q,),
                        device_id_type=pl.DeviceIdType.MESH,
                    )
                    rdma.start()

        out_ref[pl.ds(me * m_per, m_per), :] = (
            (
                y_ref[:, pl.ds(me * n_per, n_per)].astype(jnp.int32) + 128
            ).astype(jnp.float32) * deq
        )

        for d in range(1, N_DEV):
            src = lax.rem(me - d + N_DEV, N_DEV)
            recv = pltpu.make_async_remote_copy(
                src_ref=y_ref.at[:, pl.ds(0, n_per)],
                dst_ref=rx_ref.at[d],
                send_sem=send_sems.at[d],
                recv_sem=recv_sems.at[d],
                device_id=(src,),
                device_id_type=pl.DeviceIdType.MESH,
            )
            recv.wait_recv()
            out_ref[pl.ds(src * m_per, m_per), :] = (
                (rx_ref[d].astype(jnp.int32) + 128).astype(jnp.float32) * deq
            )

        for d in range(1, N_DEV):
            send = pltpu.make_async_remote_copy(
                src_ref=y_ref.at[:, pl.ds(0, n_per)],
                dst_ref=rx_ref.at[d],
                send_sem=send_sems.at[d],
                recv_sem=recv_sems.at[d],
                device_id=(0,),
                device_id_type=pl.DeviceIdType.MESH,
            )
            send.wait_send()

    return pl.pallas_call(
        body,
        out_shape=jax.ShapeDtypeStruct((m_tot, n_per), jnp.float32),
        in_specs=[
            pl.BlockSpec(memory_space=pltpu.VMEM),
            pl.BlockSpec(memory_space=pltpu.VMEM),
            pl.BlockSpec(memory_space=pltpu.SMEM),
            pl.BlockSpec(memory_space=pltpu.SMEM),
        ],
        out_specs=pl.BlockSpec(memory_space=pltpu.VMEM),
        scratch_shapes=[
            pltpu.VMEM((m_per, n), jnp.int8),
            pltpu.VMEM((N_DEV, m_per, n_per), jnp.int8),
            pltpu.SemaphoreType.DMA((N_DEV,)),
            pltpu.SemaphoreType.DMA((N_DEV,)),
        ],
        compiler_params=pltpu.CompilerParams(
            vmem_limit_bytes=100 * 1024 * 1024,
        ),
    )(x, w_mat, scale_x, scale_w)
